# baseline (speedup 1.0000x reference)
import functools

import jax
import jax.numpy as jnp
from jax import lax
from jax.experimental import pallas as pl
from jax.experimental.pallas import tpu as pltpu
from jax.experimental.pallas import tpu_sc as plsc

_NPOINT = 1024
_K = 32
_SB = 256
_PB = 2048


def _topk_body(xq_ref, xt_ref, idx_ref):
    b = pl.program_id(0)
    n = xt_ref.shape[2]
    q = xq_ref[0]
    xt = xt_ref[0]
    t = jnp.dot(q, xt, preferred_element_type=jnp.float32)
    qn = jnp.sum(q * q, axis=1, keepdims=True)
    xn = jnp.sum(xt * xt, axis=0, keepdims=True)
    d = (-2.0 * t + qn) + xn
    bits = lax.bitcast_convert_type(d, jnp.int32)
    ks = jnp.where(bits < 0, bits ^ jnp.int32(0x7FFFFFFF), bits)
    big = jnp.int32(0x7FFFFFFF)
    sb = ks.shape[0]
    ncol = n // 128
    lane = lax.broadcasted_iota(jnp.int32, (sb, 128), 1)

    chunk = lax.broadcasted_iota(jnp.int32, ks.shape, 1) >> 7
    kp = (ks & jnp.int32(~31)) | chunk

    depth = 5
    mreg = [jnp.full((sb, 128), big, jnp.int32) for _ in range(depth)]
    for c in range(ncol):
        v = kp[:, c * 128:(c + 1) * 128]
        for l in range(depth):
            lt = v < mreg[l]
            mreg[l], v = jnp.where(lt, v, mreg[l]), jnp.where(lt, mreg[l], v)

    cols = []
    kmax = jnp.zeros((sb, 1), jnp.int32)
    for _ in range(_K):
        m = jnp.min(mreg[0], axis=1, keepdims=True)
        kmax = m
        eq = mreg[0] == m
        jl = jnp.min(jnp.where(eq, lane, n), axis=1, keepdims=True)
        cols.append(((m & 31) << 7) + jl)
        pop = eq & (lane == jl)
        for l in range(depth - 1):
            mreg[l] = jnp.where(pop, mreg[l + 1], mreg[l])
        mreg[depth - 1] = jnp.where(pop, big, mreg[depth - 1])
    idx_ref[0] = jnp.concatenate(cols, axis=1) + b * n

    cnt = jnp.sum(jnp.where(kp < kmax, 1, 0).astype(jnp.int32),
                  axis=1, keepdims=True)
    bad = jnp.any(cnt > _K - 1)

    @pl.when(bad)
    def _():
        iota = lax.broadcasted_iota(jnp.int32, ks.shape, 1)
        kk = ks
        fcols = []
        for _ in range(_K):
            m = jnp.min(kk, axis=1, keepdims=True)
            j = jnp.min(jnp.where(kk == m, iota, n), axis=1, keepdims=True)
            fcols.append(j)
            kk = jnp.where(iota == j, big, kk)
        idx_ref[0] = jnp.concatenate(fcols, axis=1) + b * n


def _topk(xq, xyz_t):
    b, s, _ = xq.shape
    n = xyz_t.shape[2]
    return pl.pallas_call(
        _topk_body,
        grid=(b, s // _SB),
        in_specs=[
            pl.BlockSpec((1, _SB, 8), lambda i, j: (i, j, 0)),
            pl.BlockSpec((1, 8, n), lambda i, j: (i, 0, 0)),
        ],
        out_specs=pl.BlockSpec((1, _SB, _K), lambda i, j: (i, j, 0)),
        out_shape=jax.ShapeDtypeStruct((b, s, _K), jnp.int32),
    )(xq, xyz_t)


def _sc_gather(tbl, idx_flat):
    p = idx_flat.shape[0]
    dp = tbl.shape[1]
    info = plsc.get_sparse_core_info()
    nc, ns = info.num_cores, info.num_subcores
    nw = nc * ns
    ch = 128
    rows_w = p // nw
    nchunk = rows_w // ch
    mesh = plsc.VectorSubcoreMesh(core_axis_name="c", subcore_axis_name="s")

    @functools.partial(
        pl.kernel,
        mesh=mesh,
        out_type=jax.ShapeDtypeStruct((p, dp), tbl.dtype),
        scratch_types=[
            pltpu.VMEM((ch,), jnp.int32),
            pltpu.VMEM((ch,), jnp.int32),
            pltpu.VMEM((ch, dp), tbl.dtype),
            pltpu.VMEM((ch, dp), tbl.dtype),
            pltpu.SemaphoreType.DMA,
            pltpu.SemaphoreType.DMA,
            pltpu.SemaphoreType.DMA,
            pltpu.SemaphoreType.DMA,
            pltpu.SemaphoreType.DMA,
        ],
    )
    def gk(idx_hbm, tbl_hbm, out_hbm,
           idx_v0, idx_v1, rows_v0, rows_v1, si0, si1, sg, sw0, sw1):
        wid = lax.axis_index("s") * nc + lax.axis_index("c")
        base = wid * rows_w
        idx_v = [idx_v0, idx_v1]
        rows_v = [rows_v0, rows_v1]
        si = [si0, si1]
        sw = [sw0, sw1]
        h_idx = [None, None]
        h_w = [None, None]
        for j in range(min(2, nchunk)):
            h_idx[j] = pltpu.async_copy(
                idx_hbm.at[pl.ds(base + j * ch, ch)], idx_v[j], si[j])
        for i in range(nchunk):
            bb = i % 2
            h_idx[bb].wait()
            if h_w[bb] is not None:
                h_w[bb].wait()
            pltpu.async_copy(tbl_hbm.at[idx_v[bb]], rows_v[bb], sg).wait()
            if i + 2 < nchunk:
                h_idx[bb] = pltpu.async_copy(
                    idx_hbm.at[pl.ds(base + (i + 2) * ch, ch)], idx_v[bb], si[bb])
            h_w[bb] = pltpu.async_copy(
                rows_v[bb], out_hbm.at[pl.ds(base + i * ch, ch)], sw[bb])
        for j in range(2):
            if h_w[j] is not None:
                h_w[j].wait()

    return gk(idx_flat, tbl)


def _accum_stats(i, y, st_ref):
    s = jnp.concatenate(
        [jnp.sum(y, axis=0, keepdims=True),
         jnp.sum(y * y, axis=0, keepdims=True)], axis=0)

    @pl.when(i == 0)
    def _():
        st_ref[...] = s

    @pl.when(i != 0)
    def _():
        st_ref[...] = st_ref[...] + s


def _pass_a_body(x_ref, nx_ref, w_ref, wx_ref, b_ref, y_ref, st_ref):
    i = pl.program_id(0)
    y = jnp.dot(x_ref[...], w_ref[...], preferred_element_type=jnp.float32)
    corr = jnp.dot(nx_ref[...], wx_ref[...], preferred_element_type=jnp.float32)
    g, co = corr.shape
    corrb = jnp.broadcast_to(corr[:, None, :], (g, _K, co)).reshape(g * _K, co)
    y = (y - corrb) + b_ref[...]
    y_ref[...] = y.astype(y_ref.dtype)
    _accum_stats(i, y, st_ref)


def _pass_a(xg, nxyz, w0p, w0x, b0):
    p = xg.shape[0]
    co = w0p.shape[1]
    grid = (p // _PB,)
    gpb = _PB // _K
    return pl.pallas_call(
        _pass_a_body,
        grid=grid,
        in_specs=[
            pl.BlockSpec((_PB, xg.shape[1]), lambda i: (i, 0)),
            pl.BlockSpec((gpb, 8), lambda i: (i, 0)),
            pl.BlockSpec(w0p.shape, lambda i: (0, 0)),
            pl.BlockSpec(w0x.shape, lambda i: (0, 0)),
            pl.BlockSpec((1, co), lambda i: (0, 0)),
        ],
        out_specs=[
            pl.BlockSpec((_PB, co), lambda i: (i, 0)),
            pl.BlockSpec((2, co), lambda i: (0, 0)),
        ],
        out_shape=[
            jax.ShapeDtypeStruct((p, co), jnp.bfloat16),
            jax.ShapeDtypeStruct((2, co), jnp.float32),
        ],
    )(xg, nxyz, w0p, w0x, b0)


def _pass_bc_body(y_ref, a_ref, c_ref, w_ref, b_ref, o_ref, st_ref):
    i = pl.program_id(0)
    x = jnp.maximum(y_ref[...].astype(jnp.float32) * a_ref[...] + c_ref[...], 0.0)
    y = jnp.dot(x, w_ref[...], preferred_element_type=jnp.float32) + b_ref[...]
    o_ref[...] = y.astype(o_ref.dtype)
    _accum_stats(i, y, st_ref)


def _pass_bc(yprev, a, c, w, b, out_dtype=jnp.bfloat16):
    p, ci = yprev.shape
    co = w.shape[1]
    return pl.pallas_call(
        _pass_bc_body,
        grid=(p // _PB,),
        in_specs=[
            pl.BlockSpec((_PB, ci), lambda i: (i, 0)),
            pl.BlockSpec((1, ci), lambda i: (0, 0)),
            pl.BlockSpec((1, ci), lambda i: (0, 0)),
            pl.BlockSpec((ci, co), lambda i: (0, 0)),
            pl.BlockSpec((1, co), lambda i: (0, 0)),
        ],
        out_specs=[
            pl.BlockSpec((_PB, co), lambda i: (i, 0)),
            pl.BlockSpec((2, co), lambda i: (0, 0)),
        ],
        out_shape=[
            jax.ShapeDtypeStruct((p, co), out_dtype),
            jax.ShapeDtypeStruct((2, co), jnp.float32),
        ],
    )(yprev, a, c, w, b)


def _pass_c_body(y_ref, a_ref, c_ref, w_ref, b_ref, mx_ref, mn_ref, st_ref):
    i = pl.program_id(0)
    x = jnp.maximum(y_ref[...].astype(jnp.float32) * a_ref[...] + c_ref[...], 0.0)
    y = jnp.dot(x, w_ref[...], preferred_element_type=jnp.float32) + b_ref[...]
    g = y.shape[0] // _K
    yg = y.reshape(g, _K, y.shape[1])
    mx_ref[...] = jnp.max(yg, axis=1)
    mn_ref[...] = jnp.min(yg, axis=1)
    _accum_stats(i, y, st_ref)


def _pass_c(yprev, a, c, w, b):
    p, ci = yprev.shape
    co = w.shape[1]
    gpb = _PB // _K
    return pl.pallas_call(
        _pass_c_body,
        grid=(p // _PB,),
        in_specs=[
            pl.BlockSpec((_PB, ci), lambda i: (i, 0)),
            pl.BlockSpec((1, ci), lambda i: (0, 0)),
            pl.BlockSpec((1, ci), lambda i: (0, 0)),
            pl.BlockSpec((ci, co), lambda i: (0, 0)),
            pl.BlockSpec((1, co), lambda i: (0, 0)),
        ],
        out_specs=[
            pl.BlockSpec((gpb, co), lambda i: (i, 0)),
            pl.BlockSpec((gpb, co), lambda i: (i, 0)),
            pl.BlockSpec((2, co), lambda i: (0, 0)),
        ],
        out_shape=[
            jax.ShapeDtypeStruct((p // _K, co), jnp.float32),
            jax.ShapeDtypeStruct((p // _K, co), jnp.float32),
            jax.ShapeDtypeStruct((2, co), jnp.float32),
        ],
    )(yprev, a, c, w, b)


def _pass_d_body(mx_ref, mn_ref, a_ref, c_ref, o_ref):
    a = a_ref[...]
    hi = jnp.maximum(a * mx_ref[...] + c_ref[...], 0.0)
    lo = jnp.maximum(a * mn_ref[...] + c_ref[...], 0.0)
    o_ref[...] = jnp.where(a >= 0.0, hi, lo)


def _pass_d(mx, mn, a, c):
    q, ci = mx.shape
    qb = _PB // _K
    return pl.pallas_call(
        _pass_d_body,
        grid=(q // qb,),
        in_specs=[
            pl.BlockSpec((qb, ci), lambda i: (i, 0)),
            pl.BlockSpec((qb, ci), lambda i: (i, 0)),
            pl.BlockSpec((1, ci), lambda i: (0, 0)),
            pl.BlockSpec((1, ci), lambda i: (0, 0)),
        ],
        out_specs=pl.BlockSpec((qb, ci), lambda i: (i, 0)),
        out_shape=jax.ShapeDtypeStruct((q, ci), jnp.float32),
    )(mx, mn, a, c)


def _bn_coeffs(st, g, beta, p):
    mean = st[0] / p
    var = st[1] / p - mean * mean
    a = g / jnp.sqrt(var + 1e-5)
    c = beta - mean * a
    return a.reshape(1, -1), c.reshape(1, -1)


def kernel(xyz, points, W0, b0, g0, beta0, W1, b1, g1, beta1, W2, b2, g2, beta2):
    f32 = jnp.float32
    b, n, _ = xyz.shape
    d = points.shape[2]
    s, k = _NPOINT, _K
    p = b * s * k

    xyzp = jnp.pad(xyz, ((0, 0), (0, 0), (0, 5)))
    xyz_t = jnp.transpose(xyzp, (0, 2, 1))
    idx = _topk(xyzp[:, :s, :], xyz_t)
    idx_flat = idx.reshape(p)

    dpad = 128 - (3 + d)
    pad = jnp.zeros((b, n, dpad), f32)
    tbl = jnp.concatenate([xyz, points, pad], axis=-1).reshape(b * n, 128)
    xg = _sc_gather(tbl, idx_flat)

    nxyz = xyzp[:, :s, :].reshape(b * s, 8)
    w0p = jnp.zeros((128, W0.shape[0]), f32).at[:3 + d].set(W0.T)
    w0x = jnp.zeros((8, W0.shape[0]), f32).at[:3].set(W0[:, :3].T)

    y0, st0 = _pass_a(xg, nxyz, w0p, w0x, b0.reshape(1, -1))
    a0, c0 = _bn_coeffs(st0, g0, beta0, p)
    y1, st1 = _pass_bc(y0, a0, c0, W1.T, b1.reshape(1, -1))
    a1, c1 = _bn_coeffs(st1, g1, beta1, p)
    mx, mn, st2 = _pass_c(y1, a1, c1, W2.T, b2.reshape(1, -1))
    a2, c2 = _bn_coeffs(st2, g2, beta2, p)
    out = _pass_d(mx, mn, a2, c2)

    return xyz[:, :s, :], out.reshape(b, s, W2.shape[0])


# speedup vs baseline: 1.4586x; 1.4586x over previous
import functools

import jax
import jax.numpy as jnp
from jax import lax
from jax.experimental import pallas as pl
from jax.experimental.pallas import tpu as pltpu
from jax.experimental.pallas import tpu_sc as plsc

_NPOINT = 1024
_K = 32
_SB = 256
_PB = 2048


def _topk_body(xq_ref, xt_ref, idx_ref):
    b = pl.program_id(0)
    n = xt_ref.shape[2]
    q = xq_ref[0]
    xt = xt_ref[0]
    t = jnp.dot(q, xt, preferred_element_type=jnp.float32)
    qn = jnp.sum(q * q, axis=1, keepdims=True)
    xn = jnp.sum(xt * xt, axis=0, keepdims=True)
    d = (-2.0 * t + qn) + xn
    bits = lax.bitcast_convert_type(d, jnp.int32)
    ks = jnp.where(bits < 0, bits ^ jnp.int32(0x7FFFFFFF), bits)
    big = jnp.int32(0x7FFFFFFF)
    sb = ks.shape[0]
    ncol = n // 128
    lane = lax.broadcasted_iota(jnp.int32, (sb, 128), 1)

    depth = 6
    mreg = [jnp.full((sb, 128), big, jnp.int32) for _ in range(depth)]
    ireg = [jnp.full((sb, 128), n, jnp.int32) for _ in range(depth)]
    for c in range(ncol):
        v = ks[:, c * 128:(c + 1) * 128]
        vi = lane + (c * 128)
        for l in range(depth):
            lt = v < mreg[l]
            mreg[l], v = jnp.where(lt, v, mreg[l]), jnp.where(lt, mreg[l], v)
            ireg[l], vi = jnp.where(lt, vi, ireg[l]), jnp.where(lt, ireg[l], vi)

    cols = []
    kmax = jnp.zeros((sb, 1), jnp.int32)
    for _ in range(_K):
        m = jnp.min(mreg[0], axis=1, keepdims=True)
        kmax = m
        eq = mreg[0] == m
        j = jnp.min(jnp.where(eq, ireg[0], n), axis=1, keepdims=True)
        cols.append(j)
        pop = eq & (ireg[0] == j)
        for l in range(depth - 1):
            mreg[l] = jnp.where(pop, mreg[l + 1], mreg[l])
            ireg[l] = jnp.where(pop, ireg[l + 1], ireg[l])
        mreg[depth - 1] = jnp.where(pop, big, mreg[depth - 1])
        ireg[depth - 1] = jnp.where(pop, n, ireg[depth - 1])
    idx_ref[0] = jnp.concatenate(cols, axis=1) + b * n

    cnt = jnp.sum(jnp.where(ks < kmax, 1, 0).astype(jnp.int32),
                  axis=1, keepdims=True)
    bad = jnp.any(cnt > _K - 1)

    @pl.when(bad)
    def _():
        iota = lax.broadcasted_iota(jnp.int32, ks.shape, 1)
        kk = ks
        fcols = []
        for _ in range(_K):
            m = jnp.min(kk, axis=1, keepdims=True)
            j = jnp.min(jnp.where(kk == m, iota, n), axis=1, keepdims=True)
            fcols.append(j)
            kk = jnp.where(iota == j, big, kk)
        idx_ref[0] = jnp.concatenate(fcols, axis=1) + b * n


def _topk(xq, xyz_t):
    b, s, _ = xq.shape
    n = xyz_t.shape[2]
    return pl.pallas_call(
        _topk_body,
        grid=(b, s // _SB),
        in_specs=[
            pl.BlockSpec((1, _SB, 8), lambda i, j: (i, j, 0)),
            pl.BlockSpec((1, 8, n), lambda i, j: (i, 0, 0)),
        ],
        out_specs=pl.BlockSpec((1, _SB, _K), lambda i, j: (i, j, 0)),
        out_shape=jax.ShapeDtypeStruct((b, s, _K), jnp.int32),
    )(xq, xyz_t)


def _sc_gather(tbl, idx_flat):
    p = idx_flat.shape[0]
    dp = tbl.shape[1]
    info = plsc.get_sparse_core_info()
    nc, ns = info.num_cores, info.num_subcores
    nw = nc * ns
    ch = 128
    rows_w = p // nw
    nchunk = rows_w // ch
    mesh = plsc.VectorSubcoreMesh(core_axis_name="c", subcore_axis_name="s")

    @functools.partial(
        pl.kernel,
        mesh=mesh,
        out_type=jax.ShapeDtypeStruct((p, dp), tbl.dtype),
        scratch_types=[
            pltpu.VMEM((ch,), jnp.int32),
            pltpu.VMEM((ch,), jnp.int32),
            pltpu.VMEM((ch, dp), tbl.dtype),
            pltpu.VMEM((ch, dp), tbl.dtype),
            pltpu.SemaphoreType.DMA,
            pltpu.SemaphoreType.DMA,
            pltpu.SemaphoreType.DMA,
            pltpu.SemaphoreType.DMA,
            pltpu.SemaphoreType.DMA,
        ],
    )
    def gk(idx_hbm, tbl_hbm, out_hbm,
           idx_v0, idx_v1, rows_v0, rows_v1, si0, si1, sg, sw0, sw1):
        wid = lax.axis_index("s") * nc + lax.axis_index("c")
        base = wid * rows_w
        idx_v = [idx_v0, idx_v1]
        rows_v = [rows_v0, rows_v1]
        si = [si0, si1]
        sw = [sw0, sw1]
        h_idx = [None, None]
        h_w = [None, None]
        for j in range(min(2, nchunk)):
            h_idx[j] = pltpu.async_copy(
                idx_hbm.at[pl.ds(base + j * ch, ch)], idx_v[j], si[j])
        for i in range(nchunk):
            bb = i % 2
            h_idx[bb].wait()
            if h_w[bb] is not None:
                h_w[bb].wait()
            pltpu.async_copy(tbl_hbm.at[idx_v[bb]], rows_v[bb], sg).wait()
            if i + 2 < nchunk:
                h_idx[bb] = pltpu.async_copy(
                    idx_hbm.at[pl.ds(base + (i + 2) * ch, ch)], idx_v[bb], si[bb])
            h_w[bb] = pltpu.async_copy(
                rows_v[bb], out_hbm.at[pl.ds(base + i * ch, ch)], sw[bb])
        for j in range(2):
            if h_w[j] is not None:
                h_w[j].wait()

    return gk(idx_flat, tbl)


def _accum_stats(i, y, st_ref):
    s = jnp.concatenate(
        [jnp.sum(y, axis=0, keepdims=True),
         jnp.sum(y * y, axis=0, keepdims=True)], axis=0)

    @pl.when(i == 0)
    def _():
        st_ref[...] = s

    @pl.when(i != 0)
    def _():
        st_ref[...] = st_ref[...] + s


def _pass_a_body(x_ref, nx_ref, w_ref, wx_ref, b_ref, y_ref, st_ref):
    i = pl.program_id(0)
    y = jnp.dot(x_ref[...], w_ref[...], preferred_element_type=jnp.float32)
    corr = jnp.dot(nx_ref[...], wx_ref[...], preferred_element_type=jnp.float32)
    g, co = corr.shape
    corrb = jnp.broadcast_to(corr[:, None, :], (g, _K, co)).reshape(g * _K, co)
    y = (y - corrb) + b_ref[...]
    y_ref[...] = y.astype(y_ref.dtype)
    _accum_stats(i, y, st_ref)


def _pass_a(xg, nxyz, w0p, w0x, b0):
    p = xg.shape[0]
    co = w0p.shape[1]
    grid = (p // _PB,)
    gpb = _PB // _K
    return pl.pallas_call(
        _pass_a_body,
        grid=grid,
        in_specs=[
            pl.BlockSpec((_PB, xg.shape[1]), lambda i: (i, 0)),
            pl.BlockSpec((gpb, 8), lambda i: (i, 0)),
            pl.BlockSpec(w0p.shape, lambda i: (0, 0)),
            pl.BlockSpec(w0x.shape, lambda i: (0, 0)),
            pl.BlockSpec((1, co), lambda i: (0, 0)),
        ],
        out_specs=[
            pl.BlockSpec((_PB, co), lambda i: (i, 0)),
            pl.BlockSpec((2, co), lambda i: (0, 0)),
        ],
        out_shape=[
            jax.ShapeDtypeStruct((p, co), jnp.bfloat16),
            jax.ShapeDtypeStruct((2, co), jnp.float32),
        ],
    )(xg, nxyz, w0p, w0x, b0)


def _pass_bc_body(y_ref, a_ref, c_ref, w_ref, b_ref, o_ref, st_ref):
    i = pl.program_id(0)
    x = jnp.maximum(y_ref[...].astype(jnp.float32) * a_ref[...] + c_ref[...], 0.0)
    y = jnp.dot(x, w_ref[...], preferred_element_type=jnp.float32) + b_ref[...]
    o_ref[...] = y.astype(o_ref.dtype)
    _accum_stats(i, y, st_ref)


def _pass_bc(yprev, a, c, w, b, out_dtype=jnp.bfloat16):
    p, ci = yprev.shape
    co = w.shape[1]
    return pl.pallas_call(
        _pass_bc_body,
        grid=(p // _PB,),
        in_specs=[
            pl.BlockSpec((_PB, ci), lambda i: (i, 0)),
            pl.BlockSpec((1, ci), lambda i: (0, 0)),
            pl.BlockSpec((1, ci), lambda i: (0, 0)),
            pl.BlockSpec((ci, co), lambda i: (0, 0)),
            pl.BlockSpec((1, co), lambda i: (0, 0)),
        ],
        out_specs=[
            pl.BlockSpec((_PB, co), lambda i: (i, 0)),
            pl.BlockSpec((2, co), lambda i: (0, 0)),
        ],
        out_shape=[
            jax.ShapeDtypeStruct((p, co), out_dtype),
            jax.ShapeDtypeStruct((2, co), jnp.float32),
        ],
    )(yprev, a, c, w, b)


def _pass_c_body(y_ref, a_ref, c_ref, w_ref, b_ref, mx_ref, mn_ref, st_ref):
    i = pl.program_id(0)
    x = jnp.maximum(y_ref[...].astype(jnp.float32) * a_ref[...] + c_ref[...], 0.0)
    y = jnp.dot(x, w_ref[...], preferred_element_type=jnp.float32) + b_ref[...]
    g = y.shape[0] // _K
    yg = y.reshape(g, _K, y.shape[1])
    mx_ref[...] = jnp.max(yg, axis=1)
    mn_ref[...] = jnp.min(yg, axis=1)
    _accum_stats(i, y, st_ref)


def _pass_c(yprev, a, c, w, b):
    p, ci = yprev.shape
    co = w.shape[1]
    gpb = _PB // _K
    return pl.pallas_call(
        _pass_c_body,
        grid=(p // _PB,),
        in_specs=[
            pl.BlockSpec((_PB, ci), lambda i: (i, 0)),
            pl.BlockSpec((1, ci), lambda i: (0, 0)),
            pl.BlockSpec((1, ci), lambda i: (0, 0)),
            pl.BlockSpec((ci, co), lambda i: (0, 0)),
            pl.BlockSpec((1, co), lambda i: (0, 0)),
        ],
        out_specs=[
            pl.BlockSpec((gpb, co), lambda i: (i, 0)),
            pl.BlockSpec((gpb, co), lambda i: (i, 0)),
            pl.BlockSpec((2, co), lambda i: (0, 0)),
        ],
        out_shape=[
            jax.ShapeDtypeStruct((p // _K, co), jnp.float32),
            jax.ShapeDtypeStruct((p // _K, co), jnp.float32),
            jax.ShapeDtypeStruct((2, co), jnp.float32),
        ],
    )(yprev, a, c, w, b)


def _pass_d_body(mx_ref, mn_ref, a_ref, c_ref, o_ref):
    a = a_ref[...]
    hi = jnp.maximum(a * mx_ref[...] + c_ref[...], 0.0)
    lo = jnp.maximum(a * mn_ref[...] + c_ref[...], 0.0)
    o_ref[...] = jnp.where(a >= 0.0, hi, lo)


def _pass_d(mx, mn, a, c):
    q, ci = mx.shape
    qb = _PB // _K
    return pl.pallas_call(
        _pass_d_body,
        grid=(q // qb,),
        in_specs=[
            pl.BlockSpec((qb, ci), lambda i: (i, 0)),
            pl.BlockSpec((qb, ci), lambda i: (i, 0)),
            pl.BlockSpec((1, ci), lambda i: (0, 0)),
            pl.BlockSpec((1, ci), lambda i: (0, 0)),
        ],
        out_specs=pl.BlockSpec((qb, ci), lambda i: (i, 0)),
        out_shape=jax.ShapeDtypeStruct((q, ci), jnp.float32),
    )(mx, mn, a, c)


def _bn_coeffs(st, g, beta, p):
    mean = st[0] / p
    var = st[1] / p - mean * mean
    a = g / jnp.sqrt(var + 1e-5)
    c = beta - mean * a
    return a.reshape(1, -1), c.reshape(1, -1)


def kernel(xyz, points, W0, b0, g0, beta0, W1, b1, g1, beta1, W2, b2, g2, beta2):
    f32 = jnp.float32
    b, n, _ = xyz.shape
    d = points.shape[2]
    s, k = _NPOINT, _K
    p = b * s * k

    xyzp = jnp.pad(xyz, ((0, 0), (0, 0), (0, 5)))
    xyz_t = jnp.transpose(xyzp, (0, 2, 1))
    idx = _topk(xyzp[:, :s, :], xyz_t)
    idx_flat = idx.reshape(p)

    dpad = 128 - (3 + d)
    pad = jnp.zeros((b, n, dpad), f32)
    tbl = jnp.concatenate([xyz, points, pad], axis=-1).reshape(b * n, 128)
    xg = _sc_gather(tbl, idx_flat)

    nxyz = xyzp[:, :s, :].reshape(b * s, 8)
    w0p = jnp.zeros((128, W0.shape[0]), f32).at[:3 + d].set(W0.T)
    w0x = jnp.zeros((8, W0.shape[0]), f32).at[:3].set(W0[:, :3].T)

    y0, st0 = _pass_a(xg, nxyz, w0p, w0x, b0.reshape(1, -1))
    a0, c0 = _bn_coeffs(st0, g0, beta0, p)
    y1, st1 = _pass_bc(y0, a0, c0, W1.T, b1.reshape(1, -1))
    a1, c1 = _bn_coeffs(st1, g1, beta1, p)
    mx, mn, st2 = _pass_c(y1, a1, c1, W2.T, b2.reshape(1, -1))
    a2, c2 = _bn_coeffs(st2, g2, beta2, p)
    out = _pass_d(mx, mn, a2, c2)

    return xyz[:, :s, :], out.reshape(b, s, W2.shape[0])


# speedup vs baseline: 1.4952x; 1.0251x over previous
import functools

import jax
import jax.numpy as jnp
from jax import lax
from jax.experimental import pallas as pl
from jax.experimental.pallas import tpu as pltpu
from jax.experimental.pallas import tpu_sc as plsc

_NPOINT = 1024
_K = 32
_SB = 256
_PB = 2048


def _topk_body(xq_ref, xt_ref, idx_ref):
    b = pl.program_id(0)
    n = xt_ref.shape[2]
    q = xq_ref[0]
    xt = xt_ref[0]
    t = jnp.dot(q, xt, preferred_element_type=jnp.float32)
    qn = jnp.sum(q * q, axis=1, keepdims=True)
    xn = jnp.sum(xt * xt, axis=0, keepdims=True)
    d = (-2.0 * t + qn) + xn
    bits = lax.bitcast_convert_type(d, jnp.int32)
    ks = jnp.where(bits < 0, bits ^ jnp.int32(0x7FFFFFFF), bits)
    big = jnp.int32(0x7FFFFFFF)
    sb = ks.shape[0]
    ncol = n // 128
    lane = lax.broadcasted_iota(jnp.int32, (sb, 128), 1)

    depth = 5
    mreg = [jnp.full((sb, 128), big, jnp.int32) for _ in range(depth)]
    ireg = [jnp.full((sb, 128), n, jnp.int32) for _ in range(depth)]
    for c in range(ncol):
        v = ks[:, c * 128:(c + 1) * 128]
        vi = lane + (c * 128)
        for l in range(depth):
            lt = v < mreg[l]
            mreg[l], v = jnp.where(lt, v, mreg[l]), jnp.where(lt, mreg[l], v)
            ireg[l], vi = jnp.where(lt, vi, ireg[l]), jnp.where(lt, ireg[l], vi)

    cols = []
    kmax = jnp.zeros((sb, 1), jnp.int32)
    for _ in range(_K):
        m = jnp.min(mreg[0], axis=1, keepdims=True)
        kmax = m
        eq = mreg[0] == m
        j = jnp.min(jnp.where(eq, ireg[0], n), axis=1, keepdims=True)
        cols.append(j)
        pop = eq & (ireg[0] == j)
        for l in range(depth - 1):
            mreg[l] = jnp.where(pop, mreg[l + 1], mreg[l])
            ireg[l] = jnp.where(pop, ireg[l + 1], ireg[l])
        mreg[depth - 1] = jnp.where(pop, big, mreg[depth - 1])
        ireg[depth - 1] = jnp.where(pop, n, ireg[depth - 1])
    idx_ref[0] = jnp.concatenate(cols, axis=1) + b * n

    cnt = jnp.sum(jnp.where(ks < kmax, 1, 0).astype(jnp.int32),
                  axis=1, keepdims=True)
    bad = jnp.any(cnt > _K - 1)

    @pl.when(bad)
    def _():
        iota = lax.broadcasted_iota(jnp.int32, ks.shape, 1)
        kk = ks
        fcols = []
        for _ in range(_K):
            m = jnp.min(kk, axis=1, keepdims=True)
            j = jnp.min(jnp.where(kk == m, iota, n), axis=1, keepdims=True)
            fcols.append(j)
            kk = jnp.where(iota == j, big, kk)
        idx_ref[0] = jnp.concatenate(fcols, axis=1) + b * n


def _topk(xq, xyz_t):
    b, s, _ = xq.shape
    n = xyz_t.shape[2]
    return pl.pallas_call(
        _topk_body,
        grid=(b, s // _SB),
        in_specs=[
            pl.BlockSpec((1, _SB, 8), lambda i, j: (i, j, 0)),
            pl.BlockSpec((1, 8, n), lambda i, j: (i, 0, 0)),
        ],
        out_specs=pl.BlockSpec((1, _SB, _K), lambda i, j: (i, j, 0)),
        out_shape=jax.ShapeDtypeStruct((b, s, _K), jnp.int32),
    )(xq, xyz_t)


def _sc_gather(tbl, idx_flat):
    p = idx_flat.shape[0]
    dp = tbl.shape[1]
    info = plsc.get_sparse_core_info()
    nc, ns = info.num_cores, info.num_subcores
    nw = nc * ns
    ch = 128
    rows_w = p // nw
    nchunk = rows_w // ch
    mesh = plsc.VectorSubcoreMesh(core_axis_name="c", subcore_axis_name="s")

    @functools.partial(
        pl.kernel,
        mesh=mesh,
        out_type=jax.ShapeDtypeStruct((p, dp), tbl.dtype),
        scratch_types=[
            pltpu.VMEM((ch,), jnp.int32),
            pltpu.VMEM((ch,), jnp.int32),
            pltpu.VMEM((ch, dp), tbl.dtype),
            pltpu.VMEM((ch, dp), tbl.dtype),
            pltpu.SemaphoreType.DMA,
            pltpu.SemaphoreType.DMA,
            pltpu.SemaphoreType.DMA,
            pltpu.SemaphoreType.DMA,
            pltpu.SemaphoreType.DMA,
        ],
    )
    def gk(idx_hbm, tbl_hbm, out_hbm,
           idx_v0, idx_v1, rows_v0, rows_v1, si0, si1, sg, sw0, sw1):
        wid = lax.axis_index("s") * nc + lax.axis_index("c")
        base = wid * rows_w
        idx_v = [idx_v0, idx_v1]
        rows_v = [rows_v0, rows_v1]
        si = [si0, si1]
        sw = [sw0, sw1]
        h_idx = [None, None]
        h_w = [None, None]
        for j in range(min(2, nchunk)):
            h_idx[j] = pltpu.async_copy(
                idx_hbm.at[pl.ds(base + j * ch, ch)], idx_v[j], si[j])
        for i in range(nchunk):
            bb = i % 2
            h_idx[bb].wait()
            if h_w[bb] is not None:
                h_w[bb].wait()
            pltpu.async_copy(tbl_hbm.at[idx_v[bb]], rows_v[bb], sg).wait()
            if i + 2 < nchunk:
                h_idx[bb] = pltpu.async_copy(
                    idx_hbm.at[pl.ds(base + (i + 2) * ch, ch)], idx_v[bb], si[bb])
            h_w[bb] = pltpu.async_copy(
                rows_v[bb], out_hbm.at[pl.ds(base + i * ch, ch)], sw[bb])
        for j in range(2):
            if h_w[j] is not None:
                h_w[j].wait()

    return gk(idx_flat, tbl)


def _accum_stats(i, y, st_ref):
    s = jnp.concatenate(
        [jnp.sum(y, axis=0, keepdims=True),
         jnp.sum(y * y, axis=0, keepdims=True)], axis=0)

    @pl.when(i == 0)
    def _():
        st_ref[...] = s

    @pl.when(i != 0)
    def _():
        st_ref[...] = st_ref[...] + s


def _pass_a_body(x_ref, nx_ref, w_ref, wx_ref, b_ref, y_ref, st_ref):
    i = pl.program_id(0)
    y = jnp.dot(x_ref[...], w_ref[...], preferred_element_type=jnp.float32)
    corr = jnp.dot(nx_ref[...], wx_ref[...], preferred_element_type=jnp.float32)
    g, co = corr.shape
    corrb = jnp.broadcast_to(corr[:, None, :], (g, _K, co)).reshape(g * _K, co)
    y = (y - corrb) + b_ref[...]
    y_ref[...] = y.astype(y_ref.dtype)
    _accum_stats(i, y, st_ref)


def _pass_a(xg, nxyz, w0p, w0x, b0):
    p = xg.shape[0]
    co = w0p.shape[1]
    grid = (p // _PB,)
    gpb = _PB // _K
    return pl.pallas_call(
        _pass_a_body,
        grid=grid,
        in_specs=[
            pl.BlockSpec((_PB, xg.shape[1]), lambda i: (i, 0)),
            pl.BlockSpec((gpb, 8), lambda i: (i, 0)),
            pl.BlockSpec(w0p.shape, lambda i: (0, 0)),
            pl.BlockSpec(w0x.shape, lambda i: (0, 0)),
            pl.BlockSpec((1, co), lambda i: (0, 0)),
        ],
        out_specs=[
            pl.BlockSpec((_PB, co), lambda i: (i, 0)),
            pl.BlockSpec((2, co), lambda i: (0, 0)),
        ],
        out_shape=[
            jax.ShapeDtypeStruct((p, co), jnp.bfloat16),
            jax.ShapeDtypeStruct((2, co), jnp.float32),
        ],
    )(xg, nxyz, w0p, w0x, b0)


def _pass_bc_body(y_ref, a_ref, c_ref, w_ref, b_ref, o_ref, st_ref):
    i = pl.program_id(0)
    x = jnp.maximum(y_ref[...].astype(jnp.float32) * a_ref[...] + c_ref[...], 0.0)
    y = jnp.dot(x, w_ref[...], preferred_element_type=jnp.float32) + b_ref[...]
    o_ref[...] = y.astype(o_ref.dtype)
    _accum_stats(i, y, st_ref)


def _pass_bc(yprev, a, c, w, b, out_dtype=jnp.bfloat16):
    p, ci = yprev.shape
    co = w.shape[1]
    return pl.pallas_call(
        _pass_bc_body,
        grid=(p // _PB,),
        in_specs=[
            pl.BlockSpec((_PB, ci), lambda i: (i, 0)),
            pl.BlockSpec((1, ci), lambda i: (0, 0)),
            pl.BlockSpec((1, ci), lambda i: (0, 0)),
            pl.BlockSpec((ci, co), lambda i: (0, 0)),
            pl.BlockSpec((1, co), lambda i: (0, 0)),
        ],
        out_specs=[
            pl.BlockSpec((_PB, co), lambda i: (i, 0)),
            pl.BlockSpec((2, co), lambda i: (0, 0)),
        ],
        out_shape=[
            jax.ShapeDtypeStruct((p, co), out_dtype),
            jax.ShapeDtypeStruct((2, co), jnp.float32),
        ],
    )(yprev, a, c, w, b)


def _pass_c_body(y_ref, a_ref, c_ref, w_ref, b_ref, mx_ref, mn_ref, st_ref):
    i = pl.program_id(0)
    x = jnp.maximum(y_ref[...].astype(jnp.float32) * a_ref[...] + c_ref[...], 0.0)
    y = jnp.dot(x, w_ref[...], preferred_element_type=jnp.float32) + b_ref[...]
    g = y.shape[0] // _K
    yg = y.reshape(g, _K, y.shape[1])
    mx_ref[...] = jnp.max(yg, axis=1)
    mn_ref[...] = jnp.min(yg, axis=1)
    _accum_stats(i, y, st_ref)


def _pass_c(yprev, a, c, w, b):
    p, ci = yprev.shape
    co = w.shape[1]
    gpb = _PB // _K
    return pl.pallas_call(
        _pass_c_body,
        grid=(p // _PB,),
        in_specs=[
            pl.BlockSpec((_PB, ci), lambda i: (i, 0)),
            pl.BlockSpec((1, ci), lambda i: (0, 0)),
            pl.BlockSpec((1, ci), lambda i: (0, 0)),
            pl.BlockSpec((ci, co), lambda i: (0, 0)),
            pl.BlockSpec((1, co), lambda i: (0, 0)),
        ],
        out_specs=[
            pl.BlockSpec((gpb, co), lambda i: (i, 0)),
            pl.BlockSpec((gpb, co), lambda i: (i, 0)),
            pl.BlockSpec((2, co), lambda i: (0, 0)),
        ],
        out_shape=[
            jax.ShapeDtypeStruct((p // _K, co), jnp.float32),
            jax.ShapeDtypeStruct((p // _K, co), jnp.float32),
            jax.ShapeDtypeStruct((2, co), jnp.float32),
        ],
    )(yprev, a, c, w, b)


def _pass_d_body(mx_ref, mn_ref, a_ref, c_ref, o_ref):
    a = a_ref[...]
    hi = jnp.maximum(a * mx_ref[...] + c_ref[...], 0.0)
    lo = jnp.maximum(a * mn_ref[...] + c_ref[...], 0.0)
    o_ref[...] = jnp.where(a >= 0.0, hi, lo)


def _pass_d(mx, mn, a, c):
    q, ci = mx.shape
    qb = _PB // _K
    return pl.pallas_call(
        _pass_d_body,
        grid=(q // qb,),
        in_specs=[
            pl.BlockSpec((qb, ci), lambda i: (i, 0)),
            pl.BlockSpec((qb, ci), lambda i: (i, 0)),
            pl.BlockSpec((1, ci), lambda i: (0, 0)),
            pl.BlockSpec((1, ci), lambda i: (0, 0)),
        ],
        out_specs=pl.BlockSpec((qb, ci), lambda i: (i, 0)),
        out_shape=jax.ShapeDtypeStruct((q, ci), jnp.float32),
    )(mx, mn, a, c)


def _bn_coeffs(st, g, beta, p):
    mean = st[0] / p
    var = st[1] / p - mean * mean
    a = g / jnp.sqrt(var + 1e-5)
    c = beta - mean * a
    return a.reshape(1, -1), c.reshape(1, -1)


def kernel(xyz, points, W0, b0, g0, beta0, W1, b1, g1, beta1, W2, b2, g2, beta2):
    f32 = jnp.float32
    b, n, _ = xyz.shape
    d = points.shape[2]
    s, k = _NPOINT, _K
    p = b * s * k

    xyzp = jnp.pad(xyz, ((0, 0), (0, 0), (0, 5)))
    xyz_t = jnp.transpose(xyzp, (0, 2, 1))
    idx = _topk(xyzp[:, :s, :], xyz_t)
    idx_flat = idx.reshape(p)

    dpad = 128 - (3 + d)
    pad = jnp.zeros((b, n, dpad), f32)
    tbl = jnp.concatenate([xyz, points, pad], axis=-1).reshape(b * n, 128)
    xg = _sc_gather(tbl, idx_flat)

    nxyz = xyzp[:, :s, :].reshape(b * s, 8)
    w0p = jnp.zeros((128, W0.shape[0]), f32).at[:3 + d].set(W0.T)
    w0x = jnp.zeros((8, W0.shape[0]), f32).at[:3].set(W0[:, :3].T)

    y0, st0 = _pass_a(xg, nxyz, w0p, w0x, b0.reshape(1, -1))
    a0, c0 = _bn_coeffs(st0, g0, beta0, p)
    y1, st1 = _pass_bc(y0, a0, c0, W1.T, b1.reshape(1, -1))
    a1, c1 = _bn_coeffs(st1, g1, beta1, p)
    mx, mn, st2 = _pass_c(y1, a1, c1, W2.T, b2.reshape(1, -1))
    a2, c2 = _bn_coeffs(st2, g2, beta2, p)
    out = _pass_d(mx, mn, a2, c2)

    return xyz[:, :s, :], out.reshape(b, s, W2.shape[0])


# speedup vs baseline: 1.6927x; 1.1321x over previous
import functools

import jax
import jax.numpy as jnp
from jax import lax
from jax.experimental import pallas as pl
from jax.experimental.pallas import tpu as pltpu
from jax.experimental.pallas import tpu_sc as plsc

_NPOINT = 1024
_K = 32
_SB = 256
_PB = 4096


def _topk_body(xq_ref, xt_ref, idx_ref):
    b = pl.program_id(0)
    n = xt_ref.shape[2]
    q = xq_ref[0]
    xt = xt_ref[0]
    t = jnp.dot(q, xt, preferred_element_type=jnp.float32)
    qn = jnp.sum(q * q, axis=1, keepdims=True)
    xn = jnp.sum(xt * xt, axis=0, keepdims=True)
    d = (-2.0 * t + qn) + xn
    bits = lax.bitcast_convert_type(d, jnp.int32)
    ks = jnp.where(bits < 0, bits ^ jnp.int32(0x7FFFFFFF), bits)
    big = jnp.int32(0x7FFFFFFF)
    sb = ks.shape[0]
    ncol = n // 128
    lane = lax.broadcasted_iota(jnp.int32, (sb, 128), 1)

    depth = 5
    mreg = [jnp.full((sb, 128), big, jnp.int32) for _ in range(depth)]
    ireg = [jnp.full((sb, 128), n, jnp.int32) for _ in range(depth)]
    for c in range(ncol):
        v = ks[:, c * 128:(c + 1) * 128]
        vi = lane + (c * 128)
        for l in range(depth):
            lt = v < mreg[l]
            mreg[l], v = jnp.where(lt, v, mreg[l]), jnp.where(lt, mreg[l], v)
            ireg[l], vi = jnp.where(lt, vi, ireg[l]), jnp.where(lt, ireg[l], vi)

    cols = []
    kmax = jnp.zeros((sb, 1), jnp.int32)
    for _ in range(_K):
        m = jnp.min(mreg[0], axis=1, keepdims=True)
        kmax = m
        eq = mreg[0] == m
        j = jnp.min(jnp.where(eq, ireg[0], n), axis=1, keepdims=True)
        cols.append(j)
        pop = eq & (ireg[0] == j)
        for l in range(depth - 1):
            mreg[l] = jnp.where(pop, mreg[l + 1], mreg[l])
            ireg[l] = jnp.where(pop, ireg[l + 1], ireg[l])
        mreg[depth - 1] = jnp.where(pop, big, mreg[depth - 1])
        ireg[depth - 1] = jnp.where(pop, n, ireg[depth - 1])
    idx_ref[0] = jnp.concatenate(cols, axis=1) + b * n

    cnt = jnp.sum(jnp.where(ks < kmax, 1, 0).astype(jnp.int32),
                  axis=1, keepdims=True)
    bad = jnp.any(cnt > _K - 1)

    @pl.when(bad)
    def _():
        iota = lax.broadcasted_iota(jnp.int32, ks.shape, 1)
        kk = ks
        fcols = []
        for _ in range(_K):
            m = jnp.min(kk, axis=1, keepdims=True)
            j = jnp.min(jnp.where(kk == m, iota, n), axis=1, keepdims=True)
            fcols.append(j)
            kk = jnp.where(iota == j, big, kk)
        idx_ref[0] = jnp.concatenate(fcols, axis=1) + b * n


def _topk(xq, xyz_t):
    b, s, _ = xq.shape
    n = xyz_t.shape[2]
    return pl.pallas_call(
        _topk_body,
        grid=(b, s // _SB),
        in_specs=[
            pl.BlockSpec((1, _SB, 8), lambda i, j: (i, j, 0)),
            pl.BlockSpec((1, 8, n), lambda i, j: (i, 0, 0)),
        ],
        out_specs=pl.BlockSpec((1, _SB, _K), lambda i, j: (i, j, 0)),
        out_shape=jax.ShapeDtypeStruct((b, s, _K), jnp.int32),
    )(xq, xyz_t)


def _sc_gather(tbl, idx_flat):
    p = idx_flat.shape[0]
    dp = tbl.shape[1]
    info = plsc.get_sparse_core_info()
    nc, ns = info.num_cores, info.num_subcores
    nw = nc * ns
    ch = 128
    rows_w = p // nw
    nchunk = rows_w // ch
    mesh = plsc.VectorSubcoreMesh(core_axis_name="c", subcore_axis_name="s")

    @functools.partial(
        pl.kernel,
        mesh=mesh,
        out_type=jax.ShapeDtypeStruct((p, dp), tbl.dtype),
        scratch_types=[
            pltpu.VMEM((ch,), jnp.int32),
            pltpu.VMEM((ch,), jnp.int32),
            pltpu.VMEM((ch, dp), tbl.dtype),
            pltpu.VMEM((ch, dp), tbl.dtype),
            pltpu.SemaphoreType.DMA,
            pltpu.SemaphoreType.DMA,
            pltpu.SemaphoreType.DMA,
            pltpu.SemaphoreType.DMA,
            pltpu.SemaphoreType.DMA,
        ],
    )
    def gk(idx_hbm, tbl_hbm, out_hbm,
           idx_v0, idx_v1, rows_v0, rows_v1, si0, si1, sg, sw0, sw1):
        wid = lax.axis_index("s") * nc + lax.axis_index("c")
        base = wid * rows_w
        idx_v = [idx_v0, idx_v1]
        rows_v = [rows_v0, rows_v1]
        si = [si0, si1]
        sw = [sw0, sw1]
        h_idx = [None, None]
        h_w = [None, None]
        for j in range(min(2, nchunk)):
            h_idx[j] = pltpu.async_copy(
                idx_hbm.at[pl.ds(base + j * ch, ch)], idx_v[j], si[j])
        for i in range(nchunk):
            bb = i % 2
            h_idx[bb].wait()
            if h_w[bb] is not None:
                h_w[bb].wait()
            pltpu.async_copy(tbl_hbm.at[idx_v[bb]], rows_v[bb], sg).wait()
            if i + 2 < nchunk:
                h_idx[bb] = pltpu.async_copy(
                    idx_hbm.at[pl.ds(base + (i + 2) * ch, ch)], idx_v[bb], si[bb])
            h_w[bb] = pltpu.async_copy(
                rows_v[bb], out_hbm.at[pl.ds(base + i * ch, ch)], sw[bb])
        for j in range(2):
            if h_w[j] is not None:
                h_w[j].wait()

    return gk(idx_flat, tbl)


def _accum_stats(i, y, st_ref):
    s = jnp.concatenate(
        [jnp.sum(y, axis=0, keepdims=True),
         jnp.sum(y * y, axis=0, keepdims=True)], axis=0)

    @pl.when(i == 0)
    def _():
        st_ref[...] = s

    @pl.when(i != 0)
    def _():
        st_ref[...] = st_ref[...] + s


def _pass_a_body(x_ref, nx_ref, w_ref, wx_ref, b_ref, y_ref, st_ref):
    i = pl.program_id(0)
    y = jnp.dot(x_ref[...], w_ref[...], preferred_element_type=jnp.float32)
    corr = jnp.dot(nx_ref[...], wx_ref[...], preferred_element_type=jnp.float32)
    g, co = corr.shape
    corrb = jnp.broadcast_to(corr[:, None, :], (g, _K, co)).reshape(g * _K, co)
    y = (y - corrb) + b_ref[...]
    y_ref[...] = y.astype(y_ref.dtype)
    _accum_stats(i, y, st_ref)


def _pass_a(xg, nxyz, w0p, w0x, b0):
    p = xg.shape[0]
    co = w0p.shape[1]
    grid = (p // _PB,)
    gpb = _PB // _K
    return pl.pallas_call(
        _pass_a_body,
        grid=grid,
        in_specs=[
            pl.BlockSpec((_PB, xg.shape[1]), lambda i: (i, 0)),
            pl.BlockSpec((gpb, 8), lambda i: (i, 0)),
            pl.BlockSpec(w0p.shape, lambda i: (0, 0)),
            pl.BlockSpec(w0x.shape, lambda i: (0, 0)),
            pl.BlockSpec((1, co), lambda i: (0, 0)),
        ],
        out_specs=[
            pl.BlockSpec((_PB, co), lambda i: (i, 0)),
            pl.BlockSpec((2, co), lambda i: (0, 0)),
        ],
        out_shape=[
            jax.ShapeDtypeStruct((p, co), jnp.bfloat16),
            jax.ShapeDtypeStruct((2, co), jnp.float32),
        ],
    )(xg, nxyz, w0p, w0x, b0)


def _pass_bc_body(y_ref, a_ref, c_ref, w_ref, b_ref, o_ref, st_ref):
    i = pl.program_id(0)
    x = jnp.maximum(y_ref[...].astype(jnp.float32) * a_ref[...] + c_ref[...], 0.0)
    y = jnp.dot(x, w_ref[...], preferred_element_type=jnp.float32) + b_ref[...]
    o_ref[...] = y.astype(o_ref.dtype)
    _accum_stats(i, y, st_ref)


def _pass_bc(yprev, a, c, w, b, out_dtype=jnp.bfloat16):
    p, ci = yprev.shape
    co = w.shape[1]
    return pl.pallas_call(
        _pass_bc_body,
        grid=(p // _PB,),
        in_specs=[
            pl.BlockSpec((_PB, ci), lambda i: (i, 0)),
            pl.BlockSpec((1, ci), lambda i: (0, 0)),
            pl.BlockSpec((1, ci), lambda i: (0, 0)),
            pl.BlockSpec((ci, co), lambda i: (0, 0)),
            pl.BlockSpec((1, co), lambda i: (0, 0)),
        ],
        out_specs=[
            pl.BlockSpec((_PB, co), lambda i: (i, 0)),
            pl.BlockSpec((2, co), lambda i: (0, 0)),
        ],
        out_shape=[
            jax.ShapeDtypeStruct((p, co), out_dtype),
            jax.ShapeDtypeStruct((2, co), jnp.float32),
        ],
    )(yprev, a, c, w, b)


def _pass_c_body(y_ref, a_ref, c_ref, w_ref, b_ref, mx_ref, mn_ref, st_ref):
    i = pl.program_id(0)
    x = jnp.maximum(y_ref[...].astype(jnp.float32) * a_ref[...] + c_ref[...], 0.0)
    y = jnp.dot(x, w_ref[...], preferred_element_type=jnp.float32) + b_ref[...]
    g = y.shape[0] // _K
    yg = y.reshape(g, _K, y.shape[1])
    mx_ref[...] = jnp.max(yg, axis=1)
    mn_ref[...] = jnp.min(yg, axis=1)
    _accum_stats(i, y, st_ref)


def _pass_c(yprev, a, c, w, b):
    p, ci = yprev.shape
    co = w.shape[1]
    gpb = _PB // _K
    return pl.pallas_call(
        _pass_c_body,
        grid=(p // _PB,),
        in_specs=[
            pl.BlockSpec((_PB, ci), lambda i: (i, 0)),
            pl.BlockSpec((1, ci), lambda i: (0, 0)),
            pl.BlockSpec((1, ci), lambda i: (0, 0)),
            pl.BlockSpec((ci, co), lambda i: (0, 0)),
            pl.BlockSpec((1, co), lambda i: (0, 0)),
        ],
        out_specs=[
            pl.BlockSpec((gpb, co), lambda i: (i, 0)),
            pl.BlockSpec((gpb, co), lambda i: (i, 0)),
            pl.BlockSpec((2, co), lambda i: (0, 0)),
        ],
        out_shape=[
            jax.ShapeDtypeStruct((p // _K, co), jnp.float32),
            jax.ShapeDtypeStruct((p // _K, co), jnp.float32),
            jax.ShapeDtypeStruct((2, co), jnp.float32),
        ],
    )(yprev, a, c, w, b)


def _pass_d_body(mx_ref, mn_ref, a_ref, c_ref, o_ref):
    a = a_ref[...]
    hi = jnp.maximum(a * mx_ref[...] + c_ref[...], 0.0)
    lo = jnp.maximum(a * mn_ref[...] + c_ref[...], 0.0)
    o_ref[...] = jnp.where(a >= 0.0, hi, lo)


def _pass_d(mx, mn, a, c):
    q, ci = mx.shape
    qb = _PB // _K
    return pl.pallas_call(
        _pass_d_body,
        grid=(q // qb,),
        in_specs=[
            pl.BlockSpec((qb, ci), lambda i: (i, 0)),
            pl.BlockSpec((qb, ci), lambda i: (i, 0)),
            pl.BlockSpec((1, ci), lambda i: (0, 0)),
            pl.BlockSpec((1, ci), lambda i: (0, 0)),
        ],
        out_specs=pl.BlockSpec((qb, ci), lambda i: (i, 0)),
        out_shape=jax.ShapeDtypeStruct((q, ci), jnp.float32),
    )(mx, mn, a, c)


def _bn_coeffs(st, g, beta, p):
    mean = st[0] / p
    var = st[1] / p - mean * mean
    a = g / jnp.sqrt(var + 1e-5)
    c = beta - mean * a
    return a.reshape(1, -1), c.reshape(1, -1)


def kernel(xyz, points, W0, b0, g0, beta0, W1, b1, g1, beta1, W2, b2, g2, beta2):
    f32 = jnp.float32
    b, n, _ = xyz.shape
    d = points.shape[2]
    s, k = _NPOINT, _K
    p = b * s * k

    xyzp = jnp.pad(xyz, ((0, 0), (0, 0), (0, 5)))
    xyz_t = jnp.transpose(xyzp, (0, 2, 1))
    idx = _topk(xyzp[:, :s, :], xyz_t)
    idx_flat = idx.reshape(p)

    dpad = 128 - (3 + d)
    pad = jnp.zeros((b, n, dpad), f32)
    tbl = jnp.concatenate([xyz, points, pad], axis=-1).reshape(b * n, 128)
    xg = _sc_gather(tbl, idx_flat)

    nxyz = xyzp[:, :s, :].reshape(b * s, 8)
    w0p = jnp.zeros((128, W0.shape[0]), f32).at[:3 + d].set(W0.T)
    w0x = jnp.zeros((8, W0.shape[0]), f32).at[:3].set(W0[:, :3].T)

    y0, st0 = _pass_a(xg, nxyz, w0p, w0x, b0.reshape(1, -1))
    a0, c0 = _bn_coeffs(st0, g0, beta0, p)
    y1, st1 = _pass_bc(y0, a0, c0, W1.T, b1.reshape(1, -1))
    a1, c1 = _bn_coeffs(st1, g1, beta1, p)
    mx, mn, st2 = _pass_c(y1, a1, c1, W2.T, b2.reshape(1, -1))
    a2, c2 = _bn_coeffs(st2, g2, beta2, p)
    out = _pass_d(mx, mn, a2, c2)

    return xyz[:, :s, :], out.reshape(b, s, W2.shape[0])


# speedup vs baseline: 1.7717x; 1.0466x over previous
import functools

import jax
import jax.numpy as jnp
from jax import lax
from jax.experimental import pallas as pl
from jax.experimental.pallas import tpu as pltpu
from jax.experimental.pallas import tpu_sc as plsc

_NPOINT = 1024
_K = 32
_SB = 256
_PB = 8192


def _topk_body(xq_ref, xt_ref, idx_ref):
    b = pl.program_id(0)
    n = xt_ref.shape[2]
    q = xq_ref[0]
    xt = xt_ref[0]
    t = jnp.dot(q, xt, preferred_element_type=jnp.float32)
    qn = jnp.sum(q * q, axis=1, keepdims=True)
    xn = jnp.sum(xt * xt, axis=0, keepdims=True)
    d = (-2.0 * t + qn) + xn
    bits = lax.bitcast_convert_type(d, jnp.int32)
    ks = jnp.where(bits < 0, bits ^ jnp.int32(0x7FFFFFFF), bits)
    big = jnp.int32(0x7FFFFFFF)
    sb = ks.shape[0]
    ncol = n // 128
    lane = lax.broadcasted_iota(jnp.int32, (sb, 128), 1)

    depth = 5
    mreg = [jnp.full((sb, 128), big, jnp.int32) for _ in range(depth)]
    ireg = [jnp.full((sb, 128), n, jnp.int32) for _ in range(depth)]
    for c in range(ncol):
        v = ks[:, c * 128:(c + 1) * 128]
        vi = lane + (c * 128)
        for l in range(depth):
            lt = v < mreg[l]
            mreg[l], v = jnp.where(lt, v, mreg[l]), jnp.where(lt, mreg[l], v)
            ireg[l], vi = jnp.where(lt, vi, ireg[l]), jnp.where(lt, ireg[l], vi)

    cols = []
    kmax = jnp.zeros((sb, 1), jnp.int32)
    for _ in range(_K):
        m = jnp.min(mreg[0], axis=1, keepdims=True)
        kmax = m
        eq = mreg[0] == m
        j = jnp.min(jnp.where(eq, ireg[0], n), axis=1, keepdims=True)
        cols.append(j)
        pop = eq & (ireg[0] == j)
        for l in range(depth - 1):
            mreg[l] = jnp.where(pop, mreg[l + 1], mreg[l])
            ireg[l] = jnp.where(pop, ireg[l + 1], ireg[l])
        mreg[depth - 1] = jnp.where(pop, big, mreg[depth - 1])
        ireg[depth - 1] = jnp.where(pop, n, ireg[depth - 1])
    idx_ref[0] = jnp.concatenate(cols, axis=1) + b * n

    cnt = jnp.sum(jnp.where(ks < kmax, 1, 0).astype(jnp.int32),
                  axis=1, keepdims=True)
    bad = jnp.any(cnt > _K - 1)

    @pl.when(bad)
    def _():
        iota = lax.broadcasted_iota(jnp.int32, ks.shape, 1)
        kk = ks
        fcols = []
        for _ in range(_K):
            m = jnp.min(kk, axis=1, keepdims=True)
            j = jnp.min(jnp.where(kk == m, iota, n), axis=1, keepdims=True)
            fcols.append(j)
            kk = jnp.where(iota == j, big, kk)
        idx_ref[0] = jnp.concatenate(fcols, axis=1) + b * n


def _topk(xq, xyz_t):
    b, s, _ = xq.shape
    n = xyz_t.shape[2]
    return pl.pallas_call(
        _topk_body,
        grid=(b, s // _SB),
        in_specs=[
            pl.BlockSpec((1, _SB, 8), lambda i, j: (i, j, 0)),
            pl.BlockSpec((1, 8, n), lambda i, j: (i, 0, 0)),
        ],
        out_specs=pl.BlockSpec((1, _SB, _K), lambda i, j: (i, j, 0)),
        out_shape=jax.ShapeDtypeStruct((b, s, _K), jnp.int32),
    )(xq, xyz_t)


def _sc_gather(tbl, idx_flat):
    p = idx_flat.shape[0]
    dp = tbl.shape[1]
    info = plsc.get_sparse_core_info()
    nc, ns = info.num_cores, info.num_subcores
    nw = nc * ns
    ch = 128
    rows_w = p // nw
    nchunk = rows_w // ch
    mesh = plsc.VectorSubcoreMesh(core_axis_name="c", subcore_axis_name="s")

    @functools.partial(
        pl.kernel,
        mesh=mesh,
        out_type=jax.ShapeDtypeStruct((p, dp), tbl.dtype),
        scratch_types=[
            pltpu.VMEM((ch,), jnp.int32),
            pltpu.VMEM((ch,), jnp.int32),
            pltpu.VMEM((ch, dp), tbl.dtype),
            pltpu.VMEM((ch, dp), tbl.dtype),
            pltpu.SemaphoreType.DMA,
            pltpu.SemaphoreType.DMA,
            pltpu.SemaphoreType.DMA,
            pltpu.SemaphoreType.DMA,
            pltpu.SemaphoreType.DMA,
        ],
    )
    def gk(idx_hbm, tbl_hbm, out_hbm,
           idx_v0, idx_v1, rows_v0, rows_v1, si0, si1, sg, sw0, sw1):
        wid = lax.axis_index("s") * nc + lax.axis_index("c")
        base = wid * rows_w
        idx_v = [idx_v0, idx_v1]
        rows_v = [rows_v0, rows_v1]
        si = [si0, si1]
        sw = [sw0, sw1]
        h_idx = [None, None]
        h_w = [None, None]
        for j in range(min(2, nchunk)):
            h_idx[j] = pltpu.async_copy(
                idx_hbm.at[pl.ds(base + j * ch, ch)], idx_v[j], si[j])
        for i in range(nchunk):
            bb = i % 2
            h_idx[bb].wait()
            if h_w[bb] is not None:
                h_w[bb].wait()
            pltpu.async_copy(tbl_hbm.at[idx_v[bb]], rows_v[bb], sg).wait()
            if i + 2 < nchunk:
                h_idx[bb] = pltpu.async_copy(
                    idx_hbm.at[pl.ds(base + (i + 2) * ch, ch)], idx_v[bb], si[bb])
            h_w[bb] = pltpu.async_copy(
                rows_v[bb], out_hbm.at[pl.ds(base + i * ch, ch)], sw[bb])
        for j in range(2):
            if h_w[j] is not None:
                h_w[j].wait()

    return gk(idx_flat, tbl)


def _accum_stats(i, y, st_ref):
    s = jnp.concatenate(
        [jnp.sum(y, axis=0, keepdims=True),
         jnp.sum(y * y, axis=0, keepdims=True)], axis=0)

    @pl.when(i == 0)
    def _():
        st_ref[...] = s

    @pl.when(i != 0)
    def _():
        st_ref[...] = st_ref[...] + s


def _pass_a_body(x_ref, nx_ref, w_ref, wx_ref, b_ref, y_ref, st_ref):
    i = pl.program_id(0)
    y = jnp.dot(x_ref[...], w_ref[...], preferred_element_type=jnp.float32)
    corr = jnp.dot(nx_ref[...], wx_ref[...], preferred_element_type=jnp.float32)
    g, co = corr.shape
    corrb = jnp.broadcast_to(corr[:, None, :], (g, _K, co)).reshape(g * _K, co)
    y = (y - corrb) + b_ref[...]
    y_ref[...] = y.astype(y_ref.dtype)
    _accum_stats(i, y, st_ref)


def _pass_a(xg, nxyz, w0p, w0x, b0):
    p = xg.shape[0]
    co = w0p.shape[1]
    grid = (p // _PB,)
    gpb = _PB // _K
    return pl.pallas_call(
        _pass_a_body,
        grid=grid,
        in_specs=[
            pl.BlockSpec((_PB, xg.shape[1]), lambda i: (i, 0)),
            pl.BlockSpec((gpb, 8), lambda i: (i, 0)),
            pl.BlockSpec(w0p.shape, lambda i: (0, 0)),
            pl.BlockSpec(w0x.shape, lambda i: (0, 0)),
            pl.BlockSpec((1, co), lambda i: (0, 0)),
        ],
        out_specs=[
            pl.BlockSpec((_PB, co), lambda i: (i, 0)),
            pl.BlockSpec((2, co), lambda i: (0, 0)),
        ],
        out_shape=[
            jax.ShapeDtypeStruct((p, co), jnp.bfloat16),
            jax.ShapeDtypeStruct((2, co), jnp.float32),
        ],
    )(xg, nxyz, w0p, w0x, b0)


def _pass_bc_body(y_ref, a_ref, c_ref, w_ref, b_ref, o_ref, st_ref):
    i = pl.program_id(0)
    x = jnp.maximum(y_ref[...].astype(jnp.float32) * a_ref[...] + c_ref[...], 0.0)
    y = jnp.dot(x, w_ref[...], preferred_element_type=jnp.float32) + b_ref[...]
    o_ref[...] = y.astype(o_ref.dtype)
    _accum_stats(i, y, st_ref)


def _pass_bc(yprev, a, c, w, b, out_dtype=jnp.bfloat16):
    p, ci = yprev.shape
    co = w.shape[1]
    return pl.pallas_call(
        _pass_bc_body,
        grid=(p // _PB,),
        in_specs=[
            pl.BlockSpec((_PB, ci), lambda i: (i, 0)),
            pl.BlockSpec((1, ci), lambda i: (0, 0)),
            pl.BlockSpec((1, ci), lambda i: (0, 0)),
            pl.BlockSpec((ci, co), lambda i: (0, 0)),
            pl.BlockSpec((1, co), lambda i: (0, 0)),
        ],
        out_specs=[
            pl.BlockSpec((_PB, co), lambda i: (i, 0)),
            pl.BlockSpec((2, co), lambda i: (0, 0)),
        ],
        out_shape=[
            jax.ShapeDtypeStruct((p, co), out_dtype),
            jax.ShapeDtypeStruct((2, co), jnp.float32),
        ],
    )(yprev, a, c, w, b)


def _pass_c_body(y_ref, a_ref, c_ref, w_ref, b_ref, mx_ref, mn_ref, st_ref):
    i = pl.program_id(0)
    x = jnp.maximum(y_ref[...].astype(jnp.float32) * a_ref[...] + c_ref[...], 0.0)
    y = jnp.dot(x, w_ref[...], preferred_element_type=jnp.float32) + b_ref[...]
    g = y.shape[0] // _K
    yg = y.reshape(g, _K, y.shape[1])
    mx_ref[...] = jnp.max(yg, axis=1)
    mn_ref[...] = jnp.min(yg, axis=1)
    _accum_stats(i, y, st_ref)


def _pass_c(yprev, a, c, w, b):
    p, ci = yprev.shape
    co = w.shape[1]
    gpb = _PB // _K
    return pl.pallas_call(
        _pass_c_body,
        grid=(p // _PB,),
        in_specs=[
            pl.BlockSpec((_PB, ci), lambda i: (i, 0)),
            pl.BlockSpec((1, ci), lambda i: (0, 0)),
            pl.BlockSpec((1, ci), lambda i: (0, 0)),
            pl.BlockSpec((ci, co), lambda i: (0, 0)),
            pl.BlockSpec((1, co), lambda i: (0, 0)),
        ],
        out_specs=[
            pl.BlockSpec((gpb, co), lambda i: (i, 0)),
            pl.BlockSpec((gpb, co), lambda i: (i, 0)),
            pl.BlockSpec((2, co), lambda i: (0, 0)),
        ],
        out_shape=[
            jax.ShapeDtypeStruct((p // _K, co), jnp.float32),
            jax.ShapeDtypeStruct((p // _K, co), jnp.float32),
            jax.ShapeDtypeStruct((2, co), jnp.float32),
        ],
    )(yprev, a, c, w, b)


def _pass_d_body(mx_ref, mn_ref, a_ref, c_ref, o_ref):
    a = a_ref[...]
    hi = jnp.maximum(a * mx_ref[...] + c_ref[...], 0.0)
    lo = jnp.maximum(a * mn_ref[...] + c_ref[...], 0.0)
    o_ref[...] = jnp.where(a >= 0.0, hi, lo)


def _pass_d(mx, mn, a, c):
    q, ci = mx.shape
    qb = _PB // _K
    return pl.pallas_call(
        _pass_d_body,
        grid=(q // qb,),
        in_specs=[
            pl.BlockSpec((qb, ci), lambda i: (i, 0)),
            pl.BlockSpec((qb, ci), lambda i: (i, 0)),
            pl.BlockSpec((1, ci), lambda i: (0, 0)),
            pl.BlockSpec((1, ci), lambda i: (0, 0)),
        ],
        out_specs=pl.BlockSpec((qb, ci), lambda i: (i, 0)),
        out_shape=jax.ShapeDtypeStruct((q, ci), jnp.float32),
    )(mx, mn, a, c)


def _bn_coeffs(st, g, beta, p):
    mean = st[0] / p
    var = st[1] / p - mean * mean
    a = g / jnp.sqrt(var + 1e-5)
    c = beta - mean * a
    return a.reshape(1, -1), c.reshape(1, -1)


def kernel(xyz, points, W0, b0, g0, beta0, W1, b1, g1, beta1, W2, b2, g2, beta2):
    f32 = jnp.float32
    b, n, _ = xyz.shape
    d = points.shape[2]
    s, k = _NPOINT, _K
    p = b * s * k

    xyzp = jnp.pad(xyz, ((0, 0), (0, 0), (0, 5)))
    xyz_t = jnp.transpose(xyzp, (0, 2, 1))
    idx = _topk(xyzp[:, :s, :], xyz_t)
    idx_flat = idx.reshape(p)

    dpad = 128 - (3 + d)
    pad = jnp.zeros((b, n, dpad), f32)
    tbl = jnp.concatenate([xyz, points, pad], axis=-1).reshape(b * n, 128)
    xg = _sc_gather(tbl, idx_flat)

    nxyz = xyzp[:, :s, :].reshape(b * s, 8)
    w0p = jnp.zeros((128, W0.shape[0]), f32).at[:3 + d].set(W0.T)
    w0x = jnp.zeros((8, W0.shape[0]), f32).at[:3].set(W0[:, :3].T)

    y0, st0 = _pass_a(xg, nxyz, w0p, w0x, b0.reshape(1, -1))
    a0, c0 = _bn_coeffs(st0, g0, beta0, p)
    y1, st1 = _pass_bc(y0, a0, c0, W1.T, b1.reshape(1, -1))
    a1, c1 = _bn_coeffs(st1, g1, beta1, p)
    mx, mn, st2 = _pass_c(y1, a1, c1, W2.T, b2.reshape(1, -1))
    a2, c2 = _bn_coeffs(st2, g2, beta2, p)
    out = _pass_d(mx, mn, a2, c2)

    return xyz[:, :s, :], out.reshape(b, s, W2.shape[0])
